# baseline (speedup 1.0000x reference)
import jax
import jax.numpy as jnp
from jax.experimental import pallas as pl
from jax.experimental.pallas import tpu as pltpu

_BITS = 8
_LEVELS = float(2 ** _BITS - 1)
_HALF = float(2 ** (_BITS - 1))

_NB = 16
_W = 4


def _make_body(nb, chunk_rows):
    nsteps = 2 * nb

    def _body(x_hbm, x_ref, q_ref, ms_ref, xc_hbm, inv_ref, sem):
        p = pl.program_id(0)
        j = pl.program_id(1)
        s = p * nb + j

        def _chunk_copy(i):
            return pltpu.make_async_copy(
                x_hbm.at[pl.ds(i * chunk_rows, chunk_rows), :],
                xc_hbm.at[pl.ds(i * chunk_rows, chunk_rows), :],
                sem,
            )

        _chunk_copy(s).start()

        @pl.when(s >= _W)
        def _wait_one():
            _chunk_copy(s - _W).wait()

        @pl.when(p == 0)
        def _phase_minmax():
            bmn = jnp.min(x_ref[...])
            bmx = jnp.max(x_ref[...])

            @pl.when(j == 0)
            def _init():
                ms_ref[0] = bmn
                ms_ref[1] = bmx

            @pl.when(j != 0)
            def _acc():
                ms_ref[0] = jnp.minimum(ms_ref[0], bmn)
                ms_ref[1] = jnp.maximum(ms_ref[1], bmx)

        @pl.when(p == 1)
        def _phase_quant():
            @pl.when(j == 0)
            def _finalize():
                step = (ms_ref[1] - ms_ref[0]) / _LEVELS
                ms_ref[1] = step
                inv_ref[0] = 1.0 / step

            q_ref[...] = jnp.round(
                (x_ref[...] - ms_ref[0]) * inv_ref[0] - _HALF
            ).astype(jnp.int8)

        @pl.when(s == nsteps - 1)
        def _drain():
            for k in range(_W - 1):
                _chunk_copy(nsteps - _W + k).wait()
            _chunk_copy(nsteps - 1).wait()

    return _body


def kernel(input):
    shape = input.shape
    C = shape[-1]
    R = 1
    for s in shape[:-1]:
        R *= s
    x = input.reshape(R, C)

    nb = _NB
    bs = R // nb
    chunk_rows = R // (2 * nb)

    q, ms, xc = pl.pallas_call(
        _make_body(nb, chunk_rows),
        grid=(2, nb),
        in_specs=[
            pl.BlockSpec(memory_space=pl.ANY),
            pl.BlockSpec((bs, C), lambda p, j: (j, 0)),
        ],
        out_specs=[
            pl.BlockSpec((bs, C), lambda p, j: (jnp.where(p == 0, 0, j), 0)),
            pl.BlockSpec(memory_space=pltpu.SMEM),
            pl.BlockSpec(memory_space=pl.ANY),
        ],
        out_shape=[
            jax.ShapeDtypeStruct((R, C), jnp.int8),
            jax.ShapeDtypeStruct((2,), jnp.float32),
            jax.ShapeDtypeStruct((R, C), jnp.float32),
        ],
        scratch_shapes=[
            pltpu.SMEM((1,), jnp.float32),
            pltpu.SemaphoreType.DMA,
        ],
        compiler_params=pltpu.CompilerParams(
            dimension_semantics=("arbitrary", "arbitrary"),
        ),
    )(x, x)

    return (xc.reshape(shape), q.reshape(shape), ms)
